# baseline (speedup 1.0000x reference)
import jax
import jax.numpy as jnp
from jax import lax
from jax.experimental import pallas as pl
from jax.experimental.pallas import tpu as pltpu
from jax.experimental.pallas import tpu_sc as plsc

N = 10000
E = 320000
B = 64
D = 64

NC = 2
NS = 16
NW = NC * NS
EPW = E // NW
TI = 125
IPW = EPW // TI
OUTER = 10
IN_PER_OUTER = IPW // OUTER
CHUNK = TI * IN_PER_OUTER

_EPS_SCALE = float(1.0 / (1.0 + 1e-5) ** 0.5)


def _prep_body(x_ref, u_ref, batch_ref, wa_ref, wb_ref, wd_ref, b0_ref,
               p_ref, q_ref):
    x = x_ref[...]
    m = (batch_ref[...] == lax.broadcasted_iota(jnp.int32, (1, B), 1))
    m = m.astype(jnp.float32)
    ud = u_ref[...] @ wd_ref[...]
    p_ref[...] = x @ wa_ref[...] + m @ ud + b0_ref[...]
    q_ref[...] = x @ wb_ref[...]


def _prep(x, u, batch2, wa, wb, wd, b0):
    return pl.pallas_call(
        _prep_body,
        out_shape=[jax.ShapeDtypeStruct((N, D), jnp.float32)] * 2,
    )(x, u, batch2, wa, wb, wd, b0)


CH_G = 250


def _gather_body(p_hbm, q_hbm, row_hbm, col_hbm, s_hbm,
                 ridx_v, cidx_v, a0_v, b0_v, a1_v, b1_v, s_v,
                 sem0, sem1):
    wid = lax.axis_index("s") * NC + lax.axis_index("c")
    bufs = ((a0_v, b0_v, sem0), (a1_v, b1_v, sem1))
    n_tr = CHUNK // TI

    def fire(jj, slot):
        a_v, b_v, sem = bufs[slot]
        return (pltpu.async_copy(p_hbm.at[ridx_v.at[jj]], a_v, sem),
                pltpu.async_copy(q_hbm.at[cidx_v.at[jj]], b_v, sem))

    def outer(t, carry):
        irow = wid * IPW + t * IN_PER_OUTER
        pltpu.sync_copy(row_hbm.at[pl.ds(irow, IN_PER_OUTER)], ridx_v)
        pltpu.sync_copy(col_hbm.at[pl.ds(irow, IN_PER_OUTER)], cidx_v)
        inflight = fire(0, 0)
        for jj in range(n_tr):
            slot = jj % 2
            cur = inflight
            if jj + 1 < n_tr:
                inflight = fire(jj + 1, (jj + 1) % 2)
            cur[0].wait()
            cur[1].wait()
            a_v, b_v, _ = bufs[slot]
            j = jj % (CH_G // TI)

            def add_row(i5, c):
                for r in range(5):
                    i = i5 * 5 + r
                    base = j * TI + i
                    for k in range(4):
                        s_v[base, pl.ds(16 * k, 16)] = (
                            a_v[i, pl.ds(16 * k, 16)]
                            + b_v[i, pl.ds(16 * k, 16)])
                return c

            lax.fori_loop(0, TI // 5, add_row, 0)
            if jj % (CH_G // TI) == (CH_G // TI) - 1:
                q = jj // (CH_G // TI)
                eoff = wid * EPW + t * CHUNK + q * CH_G
                pltpu.sync_copy(s_v, s_hbm.at[pl.ds(eoff, CH_G)])
        return carry

    lax.fori_loop(0, OUTER, outer, 0)


def _sc_gather(p, q, row2d, col2d):
    fn = pl.kernel(
        _gather_body,
        out_type=jax.ShapeDtypeStruct((E, 2 * D), jnp.float32),
        compiler_params=pltpu.CompilerParams(use_tc_tiling_on_sc=False),
        mesh=plsc.VectorSubcoreMesh(core_axis_name="c", subcore_axis_name="s"),
        scratch_types=[
            pltpu.VMEM((IN_PER_OUTER, TI), jnp.int32),
            pltpu.VMEM((IN_PER_OUTER, TI), jnp.int32),
            pltpu.VMEM((TI, D), jnp.float32),
            pltpu.VMEM((TI, D), jnp.float32),
            pltpu.VMEM((TI, D), jnp.float32),
            pltpu.VMEM((TI, D), jnp.float32),
            pltpu.VMEM((CH_G, 2 * D), jnp.float32),
            pltpu.SemaphoreType.DMA,
            pltpu.SemaphoreType.DMA,
        ],
    )
    return fn(p, q, row2d, col2d)


def _edge_body(s_ref, eat_ref, w0_ref, w1_ref, w2_ref,
               s0_ref, t0_ref, b1_ref, s1_ref, t1_ref, b2_ref, s2_ref, t2_ref,
               et_ref, ew_ref):
    st = jnp.transpose(s_ref[:, 0:D])
    h = w0_ref[...] @ eat_ref[...] + st
    h = jnp.maximum(h, 0.0) * s0_ref[...] + t0_ref[...]
    h = w1_ref[...] @ h + b1_ref[...]
    h = jnp.maximum(h, 0.0) * s1_ref[...] + t1_ref[...]
    h = w2_ref[...] @ h + b2_ref[...]
    h = jnp.maximum(h, 0.0) * s2_ref[...] + t2_ref[...]
    et_ref[...] = h
    hr = jnp.transpose(h)
    ew_ref[:, 0:D] = hr
    ew_ref[:, D:D + 16] = jnp.full((hr.shape[0], 16), 1.0, jnp.float32)
    ew_ref[:, D + 16:2 * D] = jnp.zeros((hr.shape[0], 48), jnp.float32)


def _edge_mlp(s, ea_t, w0t, w1t, w2t, s0, t0, b1, s1, t1, b2, s2, t2):
    blk = 2560
    grid = E // blk
    s_spec = pl.BlockSpec((blk, 2 * D), lambda i: (i, 0))
    t_spec = pl.BlockSpec((D, blk), lambda i: (0, i))
    w_spec = pl.BlockSpec((D, D), lambda i: (0, 0))
    v_spec = pl.BlockSpec((D, 1), lambda i: (0, 0))
    ew_spec = pl.BlockSpec((blk, 2 * D), lambda i: (i, 0))
    return pl.pallas_call(
        _edge_body,
        grid=(grid,),
        in_specs=[s_spec, t_spec,
                  w_spec, w_spec, w_spec,
                  v_spec, v_spec, v_spec, v_spec, v_spec, v_spec, v_spec,
                  v_spec],
        out_specs=[t_spec, ew_spec],
        out_shape=[jax.ShapeDtypeStruct((D, E), jnp.float32),
                   jax.ShapeDtypeStruct((E, 2 * D), jnp.float32)],
    )(s, ea_t, w0t, w1t, w2t, s0, t0, b1, s1, t1, b2, s2, t2)


CH_S = 125


def _scatter_body(ew_hbm, row_hbm, zn_hbm, sums_hbm,
                  ridx_v, rows_v, rows1_v, shared_sum, sem0, sem1):
    cid = lax.axis_index("c")
    sid = lax.axis_index("s")
    wid = sid * NC + cid
    npt = N // NS

    pltpu.sync_copy(zn_hbm.at[pl.ds(sid * npt, npt)],
                    shared_sum.at[pl.ds(sid * npt, npt)])
    plsc.subcore_barrier()

    nq = CHUNK // CH_S

    def load(t, q, slot):
        eoff = wid * EPW + t * CHUNK + q * CH_S
        buf, sem = (rows_v, sem0) if slot == 0 else (rows1_v, sem1)
        return pltpu.async_copy(ew_hbm.at[pl.ds(eoff, CH_S)], buf, sem)

    def outer(t, carry):
        irow = wid * IPW + t * IN_PER_OUTER
        pltpu.sync_copy(row_hbm.at[pl.ds(irow, IN_PER_OUTER)], ridx_v)
        inflight = load(t, 0, 0)
        for q in range(nq):
            slot = q % 2
            cur = inflight
            if q + 1 < nq:
                inflight = load(t, q + 1, (q + 1) % 2)
            cur.wait()
            buf = rows_v if slot == 0 else rows1_v
            for j in range(CH_S // TI):
                jj = q * (CH_S // TI) + j
                pltpu.sync_copy(buf.at[pl.ds(j * TI, TI)],
                                shared_sum.at[ridx_v.at[jj]], add=True)
        return carry

    lax.fori_loop(0, OUTER, outer, 0)
    plsc.subcore_barrier()

    pltpu.sync_copy(shared_sum.at[pl.ds(sid * npt, npt)],
                    sums_hbm.at[pl.ds(cid * N + sid * npt, npt)])


def _sc_scatter(ewide, row2d, zn):
    fn = pl.kernel(
        _scatter_body,
        out_type=jax.ShapeDtypeStruct((NC * N, 2 * D), jnp.float32),
        compiler_params=pltpu.CompilerParams(use_tc_tiling_on_sc=False),
        mesh=plsc.VectorSubcoreMesh(core_axis_name="c", subcore_axis_name="s"),
        scratch_types=[
            pltpu.VMEM((IN_PER_OUTER, TI), jnp.int32),
            pltpu.VMEM((CH_S, 2 * D), jnp.float32),
            pltpu.VMEM((CH_S, 2 * D), jnp.float32),
            pltpu.VMEM_SHARED((N, 2 * D), jnp.float32),
            pltpu.SemaphoreType.DMA,
            pltpu.SemaphoreType.DMA,
        ],
    )
    return fn(ewide, row2d, zn)


def _final_body(x_ref, u_ref, batch2_ref, batchr_ref, sa_ref, sb_ref,
                nwx_ref, nwv_ref, nwu_ref, nw1_ref, nw2_ref,
                nb0_ref, ns0_ref, nt0_ref, nb1_ref, ns1_ref, nt1_ref,
                nb2_ref, ns2_ref, nt2_ref,
                gwe_ref, gwv_ref, gwu_ref, gw1_ref, gw2_ref,
                gb0_ref, gs0_ref, gt0_ref, gb1_ref, gs1_ref, gt1_ref,
                gb2_ref, gs2_ref, gt2_ref,
                xout_ref, uout_ref):
    cnt = sa_ref[:, D:D + 1] + sb_ref[:, D:D + 1]
    v_e = ((sa_ref[:, 0:D] + sb_ref[:, 0:D])
           / jnp.maximum(cnt, 1.0))
    x = x_ref[...]
    uu = u_ref[...]
    m = (batch2_ref[...] == lax.broadcasted_iota(jnp.int32, (1, B), 1))
    m = m.astype(jnp.float32)
    mt = (lax.broadcasted_iota(jnp.int32, (B, 1), 0) == batchr_ref[0:1, :])
    mt = mt.astype(jnp.float32)

    h = x @ nwx_ref[...] + v_e @ nwv_ref[...] + m @ (uu @ nwu_ref[...])
    h = h + nb0_ref[...]
    h = jnp.maximum(h, 0.0) * ns0_ref[...] + nt0_ref[...]
    h = h @ nw1_ref[...] + nb1_ref[...]
    h = jnp.maximum(h, 0.0) * ns1_ref[...] + nt1_ref[...]
    h = h @ nw2_ref[...] + nb2_ref[...]
    h = jnp.maximum(h, 0.0) * ns2_ref[...] + nt2_ref[...]
    xout_ref[...] = h

    cnt_g = jnp.maximum(jnp.sum(mt, axis=1, keepdims=True), 1.0)
    u_e = (mt @ v_e) / cnt_g
    u_v = (mt @ h) / cnt_g
    g = u_e @ gwe_ref[...] + u_v @ gwv_ref[...] + uu @ gwu_ref[...]
    g = g + gb0_ref[...]
    g = jnp.maximum(g, 0.0) * gs0_ref[...] + gt0_ref[...]
    g = g @ gw1_ref[...] + gb1_ref[...]
    g = jnp.maximum(g, 0.0) * gs1_ref[...] + gt1_ref[...]
    g = g @ gw2_ref[...] + gb2_ref[...]
    g = jnp.maximum(g, 0.0) * gs2_ref[...] + gt2_ref[...]
    uout_ref[...] = g


def _final(args):
    return pl.pallas_call(
        _final_body,
        out_shape=[jax.ShapeDtypeStruct((N, D), jnp.float32),
                   jax.ShapeDtypeStruct((B, D), jnp.float32)],
    )(*args)


def kernel(x, edge_attr, u, edge_index, batch,
           eW0, eWr, eb, eg, ebe,
           nW0, nWr, nb, ng, nbe,
           gW0, gWr, gb, gg, gbe):
    row = edge_index[0]
    col = edge_index[1]
    batch2 = batch[:, None]
    batchr = jnp.broadcast_to(batch[None, :], (8, N))

    p, q = _prep(x, u, batch2,
                 eW0[0:D], eW0[D:2 * D], eW0[3 * D:4 * D],
                 eb[0].reshape(1, D))

    row2d = row.reshape(E // TI, TI)
    col2d = col.reshape(E // TI, TI)
    s = _sc_gather(p, q, row2d, col2d)

    def r1(v):
        return v.reshape(1, D)

    def c1(v):
        return v.reshape(D, 1)
    e_t, ewide = _edge_mlp(
        s, edge_attr.T,
        eW0[2 * D:3 * D].T, eWr[0].T, eWr[1].T,
        c1(eg[0] * _EPS_SCALE), c1(ebe[0]),
        c1(eb[1]), c1(eg[1] * _EPS_SCALE), c1(ebe[1]),
        c1(eb[2]), c1(eg[2] * _EPS_SCALE), c1(ebe[2]))
    e_out = e_t.T

    sums = _sc_scatter(ewide, row2d, jnp.zeros((N, 2 * D), jnp.float32))

    x_out, u_out = _final([
        x, u, batch2, batchr,
        sums[0:N], sums[N:2 * N],
        nW0[0:D], nW0[D:2 * D], nW0[2 * D:3 * D], nWr[0], nWr[1],
        r1(nb[0]), r1(ng[0] * _EPS_SCALE), r1(nbe[0]),
        r1(nb[1]), r1(ng[1] * _EPS_SCALE), r1(nbe[1]),
        r1(nb[2]), r1(ng[2] * _EPS_SCALE), r1(nbe[2]),
        gW0[0:D], gW0[D:2 * D], gW0[2 * D:3 * D], gWr[0], gWr[1],
        r1(gb[0]), r1(gg[0] * _EPS_SCALE), r1(gbe[0]),
        r1(gb[1]), r1(gg[1] * _EPS_SCALE), r1(gbe[1]),
        r1(gb[2]), r1(gg[2] * _EPS_SCALE), r1(gbe[2]),
    ])
    return x_out, e_out, u_out


# speedup vs baseline: 1.1293x; 1.1293x over previous
import jax
import jax.numpy as jnp
from jax import lax
from jax.experimental import pallas as pl
from jax.experimental.pallas import tpu as pltpu
from jax.experimental.pallas import tpu_sc as plsc

N = 10000
E = 320000
B = 64
D = 64

NC = 2
NS = 16
NW = NC * NS
EPW = E // NW
TI = 125
IPW = EPW // TI
OUTER = 10
IN_PER_OUTER = IPW // OUTER
CHUNK = TI * IN_PER_OUTER

_EPS_SCALE = float(1.0 / (1.0 + 1e-5) ** 0.5)


def _prep_body(x_ref, u_ref, batch_ref, wa_ref, wb_ref, wd_ref, b0_ref,
               p_ref):
    x = x_ref[...]
    m = (batch_ref[...] == lax.broadcasted_iota(jnp.int32, (1, B), 1))
    m = m.astype(jnp.float32)
    ud = u_ref[...] @ wd_ref[...]
    p_ref[:, 0:D] = x @ wa_ref[...] + m @ ud + b0_ref[...]
    p_ref[:, D:2 * D] = x @ wb_ref[...]


def _prep(x, u, batch2, wa, wb, wd, b0):
    return pl.pallas_call(
        _prep_body,
        out_shape=jax.ShapeDtypeStruct((N, 2 * D), jnp.float32),
    )(x, u, batch2, wa, wb, wd, b0)


CH_G = 250


def _gather_body(pq_hbm, row_hbm, col_hbm, s_hbm,
                 ridx_v, cidx_v, a0_v, b0_v, a1_v, b1_v, s_v,
                 sem0, sem1):
    wid = lax.axis_index("s") * NC + lax.axis_index("c")
    bufs = ((a0_v, b0_v, sem0), (a1_v, b1_v, sem1))
    n_tr = CHUNK // TI

    def fire(jj, slot):
        a_v, b_v, sem = bufs[slot]
        return (pltpu.async_copy(pq_hbm.at[ridx_v.at[jj]], a_v, sem),
                pltpu.async_copy(pq_hbm.at[cidx_v.at[jj]], b_v, sem))

    def outer(t, carry):
        irow = wid * IPW + t * IN_PER_OUTER
        pltpu.sync_copy(row_hbm.at[pl.ds(irow, IN_PER_OUTER)], ridx_v)
        pltpu.sync_copy(col_hbm.at[pl.ds(irow, IN_PER_OUTER)], cidx_v)
        inflight = fire(0, 0)
        for jj in range(n_tr):
            slot = jj % 2
            cur = inflight
            if jj + 1 < n_tr:
                inflight = fire(jj + 1, (jj + 1) % 2)
            cur[0].wait()
            cur[1].wait()
            a_v, b_v, _ = bufs[slot]
            j = jj % (CH_G // TI)

            def add_row(i5, c):
                for r in range(5):
                    i = i5 * 5 + r
                    base = j * TI + i
                    for k in range(4):
                        s_v[base, pl.ds(16 * k, 16)] = (
                            a_v[i, pl.ds(16 * k, 16)]
                            + b_v[i, pl.ds(D + 16 * k, 16)])
                return c

            lax.fori_loop(0, TI // 5, add_row, 0)
            if jj % (CH_G // TI) == (CH_G // TI) - 1:
                q = jj // (CH_G // TI)
                eoff = wid * EPW + t * CHUNK + q * CH_G
                pltpu.sync_copy(s_v, s_hbm.at[pl.ds(eoff, CH_G)])
        return carry

    lax.fori_loop(0, OUTER, outer, 0)


def _sc_gather(pq, row2d, col2d):
    fn = pl.kernel(
        _gather_body,
        out_type=jax.ShapeDtypeStruct((E, 2 * D), jnp.float32),
        compiler_params=pltpu.CompilerParams(use_tc_tiling_on_sc=False),
        mesh=plsc.VectorSubcoreMesh(core_axis_name="c", subcore_axis_name="s"),
        scratch_types=[
            pltpu.VMEM((IN_PER_OUTER, TI), jnp.int32),
            pltpu.VMEM((IN_PER_OUTER, TI), jnp.int32),
            pltpu.VMEM((TI, 2 * D), jnp.float32),
            pltpu.VMEM((TI, 2 * D), jnp.float32),
            pltpu.VMEM((TI, 2 * D), jnp.float32),
            pltpu.VMEM((TI, 2 * D), jnp.float32),
            pltpu.VMEM((CH_G, 2 * D), jnp.float32),
            pltpu.SemaphoreType.DMA,
            pltpu.SemaphoreType.DMA,
        ],
    )
    return fn(pq, row2d, col2d)


def _edge_body(s_ref, eat_ref, w0_ref, w1_ref, w2_ref,
               s0_ref, t0_ref, b1_ref, s1_ref, t1_ref, b2_ref, s2_ref, t2_ref,
               et_ref, ew_ref):
    st = jnp.transpose(s_ref[:, 0:D])
    h = w0_ref[...] @ eat_ref[...] + st
    h = jnp.maximum(h, 0.0) * s0_ref[...] + t0_ref[...]
    h = w1_ref[...] @ h + b1_ref[...]
    h = jnp.maximum(h, 0.0) * s1_ref[...] + t1_ref[...]
    h = w2_ref[...] @ h + b2_ref[...]
    h = jnp.maximum(h, 0.0) * s2_ref[...] + t2_ref[...]
    et_ref[...] = h
    hr = jnp.transpose(h)
    ew_ref[:, 0:D] = hr
    ew_ref[:, D:D + 16] = jnp.full((hr.shape[0], 16), 1.0, jnp.float32)
    ew_ref[:, D + 16:2 * D] = jnp.zeros((hr.shape[0], 48), jnp.float32)


def _edge_mlp(s, ea_t, w0t, w1t, w2t, s0, t0, b1, s1, t1, b2, s2, t2):
    blk = 2560
    grid = E // blk
    s_spec = pl.BlockSpec((blk, 2 * D), lambda i: (i, 0))
    t_spec = pl.BlockSpec((D, blk), lambda i: (0, i))
    w_spec = pl.BlockSpec((D, D), lambda i: (0, 0))
    v_spec = pl.BlockSpec((D, 1), lambda i: (0, 0))
    ew_spec = pl.BlockSpec((blk, 2 * D), lambda i: (i, 0))
    return pl.pallas_call(
        _edge_body,
        grid=(grid,),
        in_specs=[s_spec, t_spec,
                  w_spec, w_spec, w_spec,
                  v_spec, v_spec, v_spec, v_spec, v_spec, v_spec, v_spec,
                  v_spec],
        out_specs=[t_spec, ew_spec],
        out_shape=[jax.ShapeDtypeStruct((D, E), jnp.float32),
                   jax.ShapeDtypeStruct((E, 2 * D), jnp.float32)],
    )(s, ea_t, w0t, w1t, w2t, s0, t0, b1, s1, t1, b2, s2, t2)


CH_S = 125


def _scatter_body(ew_hbm, row_hbm, zn_hbm, sums_hbm,
                  ridx_v, rows_v, rows1_v, shared_sum, sem0, sem1):
    cid = lax.axis_index("c")
    sid = lax.axis_index("s")
    wid = sid * NC + cid
    npt = N // NS

    pltpu.sync_copy(zn_hbm.at[pl.ds(sid * npt, npt)],
                    shared_sum.at[pl.ds(sid * npt, npt)])
    plsc.subcore_barrier()

    nq = CHUNK // CH_S

    def load(t, q, slot):
        eoff = wid * EPW + t * CHUNK + q * CH_S
        buf, sem = (rows_v, sem0) if slot == 0 else (rows1_v, sem1)
        return pltpu.async_copy(ew_hbm.at[pl.ds(eoff, CH_S)], buf, sem)

    def outer(t, carry):
        irow = wid * IPW + t * IN_PER_OUTER
        pltpu.sync_copy(row_hbm.at[pl.ds(irow, IN_PER_OUTER)], ridx_v)
        inflight = load(t, 0, 0)
        for q in range(nq):
            slot = q % 2
            cur = inflight
            if q + 1 < nq:
                inflight = load(t, q + 1, (q + 1) % 2)
            cur.wait()
            buf = rows_v if slot == 0 else rows1_v
            for j in range(CH_S // TI):
                jj = q * (CH_S // TI) + j
                pltpu.sync_copy(buf.at[pl.ds(j * TI, TI)],
                                shared_sum.at[ridx_v.at[jj]], add=True)
        return carry

    lax.fori_loop(0, OUTER, outer, 0)
    plsc.subcore_barrier()

    pltpu.sync_copy(shared_sum.at[pl.ds(sid * npt, npt)],
                    sums_hbm.at[pl.ds(cid * N + sid * npt, npt)])


def _sc_scatter(ewide, row2d, zn):
    fn = pl.kernel(
        _scatter_body,
        out_type=jax.ShapeDtypeStruct((NC * N, 2 * D), jnp.float32),
        compiler_params=pltpu.CompilerParams(use_tc_tiling_on_sc=False),
        mesh=plsc.VectorSubcoreMesh(core_axis_name="c", subcore_axis_name="s"),
        scratch_types=[
            pltpu.VMEM((IN_PER_OUTER, TI), jnp.int32),
            pltpu.VMEM((CH_S, 2 * D), jnp.float32),
            pltpu.VMEM((CH_S, 2 * D), jnp.float32),
            pltpu.VMEM_SHARED((N, 2 * D), jnp.float32),
            pltpu.SemaphoreType.DMA,
            pltpu.SemaphoreType.DMA,
        ],
    )
    return fn(ewide, row2d, zn)


def _final_body(x_ref, u_ref, batch2_ref, batchr_ref, sa_ref, sb_ref,
                nwx_ref, nwv_ref, nwu_ref, nw1_ref, nw2_ref,
                nb0_ref, ns0_ref, nt0_ref, nb1_ref, ns1_ref, nt1_ref,
                nb2_ref, ns2_ref, nt2_ref,
                gwe_ref, gwv_ref, gwu_ref, gw1_ref, gw2_ref,
                gb0_ref, gs0_ref, gt0_ref, gb1_ref, gs1_ref, gt1_ref,
                gb2_ref, gs2_ref, gt2_ref,
                xout_ref, uout_ref):
    cnt = sa_ref[:, D:D + 1] + sb_ref[:, D:D + 1]
    v_e = ((sa_ref[:, 0:D] + sb_ref[:, 0:D])
           / jnp.maximum(cnt, 1.0))
    x = x_ref[...]
    uu = u_ref[...]
    m = (batch2_ref[...] == lax.broadcasted_iota(jnp.int32, (1, B), 1))
    m = m.astype(jnp.float32)
    mt = (lax.broadcasted_iota(jnp.int32, (B, 1), 0) == batchr_ref[0:1, :])
    mt = mt.astype(jnp.float32)

    h = x @ nwx_ref[...] + v_e @ nwv_ref[...] + m @ (uu @ nwu_ref[...])
    h = h + nb0_ref[...]
    h = jnp.maximum(h, 0.0) * ns0_ref[...] + nt0_ref[...]
    h = h @ nw1_ref[...] + nb1_ref[...]
    h = jnp.maximum(h, 0.0) * ns1_ref[...] + nt1_ref[...]
    h = h @ nw2_ref[...] + nb2_ref[...]
    h = jnp.maximum(h, 0.0) * ns2_ref[...] + nt2_ref[...]
    xout_ref[...] = h

    cnt_g = jnp.maximum(jnp.sum(mt, axis=1, keepdims=True), 1.0)
    u_e = (mt @ v_e) / cnt_g
    u_v = (mt @ h) / cnt_g
    g = u_e @ gwe_ref[...] + u_v @ gwv_ref[...] + uu @ gwu_ref[...]
    g = g + gb0_ref[...]
    g = jnp.maximum(g, 0.0) * gs0_ref[...] + gt0_ref[...]
    g = g @ gw1_ref[...] + gb1_ref[...]
    g = jnp.maximum(g, 0.0) * gs1_ref[...] + gt1_ref[...]
    g = g @ gw2_ref[...] + gb2_ref[...]
    g = jnp.maximum(g, 0.0) * gs2_ref[...] + gt2_ref[...]
    uout_ref[...] = g


def _final(args):
    return pl.pallas_call(
        _final_body,
        out_shape=[jax.ShapeDtypeStruct((N, D), jnp.float32),
                   jax.ShapeDtypeStruct((B, D), jnp.float32)],
    )(*args)


def kernel(x, edge_attr, u, edge_index, batch,
           eW0, eWr, eb, eg, ebe,
           nW0, nWr, nb, ng, nbe,
           gW0, gWr, gb, gg, gbe):
    row = edge_index[0]
    col = edge_index[1]
    batch2 = batch[:, None]
    batchr = jnp.broadcast_to(batch[None, :], (8, N))

    pq = _prep(x, u, batch2,
               eW0[0:D], eW0[D:2 * D], eW0[3 * D:4 * D],
               eb[0].reshape(1, D))

    row2d = row.reshape(E // TI, TI)
    col2d = col.reshape(E // TI, TI)
    s = _sc_gather(pq, row2d, col2d)

    def r1(v):
        return v.reshape(1, D)

    def c1(v):
        return v.reshape(D, 1)
    e_t, ewide = _edge_mlp(
        s, edge_attr.T,
        eW0[2 * D:3 * D].T, eWr[0].T, eWr[1].T,
        c1(eg[0] * _EPS_SCALE), c1(ebe[0]),
        c1(eb[1]), c1(eg[1] * _EPS_SCALE), c1(ebe[1]),
        c1(eb[2]), c1(eg[2] * _EPS_SCALE), c1(ebe[2]))
    e_out = e_t.T

    sums = _sc_scatter(ewide, row2d, jnp.zeros((N, 2 * D), jnp.float32))

    x_out, u_out = _final([
        x, u, batch2, batchr,
        sums[0:N], sums[N:2 * N],
        nW0[0:D], nW0[D:2 * D], nW0[2 * D:3 * D], nWr[0], nWr[1],
        r1(nb[0]), r1(ng[0] * _EPS_SCALE), r1(nbe[0]),
        r1(nb[1]), r1(ng[1] * _EPS_SCALE), r1(nbe[1]),
        r1(nb[2]), r1(ng[2] * _EPS_SCALE), r1(nbe[2]),
        gW0[0:D], gW0[D:2 * D], gW0[2 * D:3 * D], gWr[0], gWr[1],
        r1(gb[0]), r1(gg[0] * _EPS_SCALE), r1(gbe[0]),
        r1(gb[1]), r1(gg[1] * _EPS_SCALE), r1(gbe[1]),
        r1(gb[2]), r1(gg[2] * _EPS_SCALE), r1(gbe[2]),
    ])
    return x_out, e_out, u_out


# speedup vs baseline: 1.2285x; 1.0878x over previous
import jax
import jax.numpy as jnp
from jax import lax
from jax.experimental import pallas as pl
from jax.experimental.pallas import tpu as pltpu
from jax.experimental.pallas import tpu_sc as plsc

N = 10000
E = 320000
B = 64
D = 64

NC = 2
NS = 16
NW = NC * NS
NCH = 2
EH = E // NCH
EPW = EH // NW
TI = 125
IPW = EPW // TI
OUTER = 5
IN_PER_OUTER = IPW // OUTER
CHUNK = TI * IN_PER_OUTER

_EPS_SCALE = float(1.0 / (1.0 + 1e-5) ** 0.5)


def _prep_body(x_ref, u_ref, batch_ref, wa_ref, wb_ref, wd_ref, b0_ref,
               p_ref):
    x = x_ref[...]
    m = (batch_ref[...] == lax.broadcasted_iota(jnp.int32, (1, B), 1))
    m = m.astype(jnp.float32)
    ud = u_ref[...] @ wd_ref[...]
    p_ref[:, 0:D] = x @ wa_ref[...] + m @ ud + b0_ref[...]
    p_ref[:, D:2 * D] = x @ wb_ref[...]


def _prep(x, u, batch2, wa, wb, wd, b0):
    return pl.pallas_call(
        _prep_body,
        out_shape=jax.ShapeDtypeStruct((N, 2 * D), jnp.float32),
    )(x, u, batch2, wa, wb, wd, b0)


CH_G = 250


def _gather_body(pq_hbm, row_hbm, col_hbm, s_hbm,
                 ridx_v, cidx_v, a0_v, b0_v, a1_v, b1_v, s_v,
                 sem0, sem1):
    wid = lax.axis_index("s") * NC + lax.axis_index("c")
    bufs = ((a0_v, b0_v, sem0), (a1_v, b1_v, sem1))
    n_tr = CHUNK // TI

    def fire(jj, slot):
        a_v, b_v, sem = bufs[slot]
        return (pltpu.async_copy(pq_hbm.at[ridx_v.at[jj]], a_v, sem),
                pltpu.async_copy(pq_hbm.at[cidx_v.at[jj]], b_v, sem))

    def outer(t, carry):
        irow = wid * IPW + t * IN_PER_OUTER
        pltpu.sync_copy(row_hbm.at[pl.ds(irow, IN_PER_OUTER)], ridx_v)
        pltpu.sync_copy(col_hbm.at[pl.ds(irow, IN_PER_OUTER)], cidx_v)
        inflight = fire(0, 0)
        for jj in range(n_tr):
            slot = jj % 2
            cur = inflight
            if jj + 1 < n_tr:
                inflight = fire(jj + 1, (jj + 1) % 2)
            cur[0].wait()
            cur[1].wait()
            a_v, b_v, _ = bufs[slot]
            j = jj % (CH_G // TI)

            def add_row(i5, c):
                for r in range(5):
                    i = i5 * 5 + r
                    base = j * TI + i
                    for k in range(4):
                        s_v[base, pl.ds(16 * k, 16)] = (
                            a_v[i, pl.ds(16 * k, 16)]
                            + b_v[i, pl.ds(D + 16 * k, 16)])
                return c

            lax.fori_loop(0, TI // 5, add_row, 0)
            if jj % (CH_G // TI) == (CH_G // TI) - 1:
                q = jj // (CH_G // TI)
                eoff = wid * EPW + t * CHUNK + q * CH_G
                pltpu.sync_copy(s_v, s_hbm.at[pl.ds(eoff, CH_G)])
        return carry

    lax.fori_loop(0, OUTER, outer, 0)


def _sc_gather(pq, row2d, col2d):
    fn = pl.kernel(
        _gather_body,
        out_type=jax.ShapeDtypeStruct((EH, 2 * D), jnp.float32),
        compiler_params=pltpu.CompilerParams(use_tc_tiling_on_sc=False),
        mesh=plsc.VectorSubcoreMesh(core_axis_name="c", subcore_axis_name="s"),
        scratch_types=[
            pltpu.VMEM((IN_PER_OUTER, TI), jnp.int32),
            pltpu.VMEM((IN_PER_OUTER, TI), jnp.int32),
            pltpu.VMEM((TI, 2 * D), jnp.float32),
            pltpu.VMEM((TI, 2 * D), jnp.float32),
            pltpu.VMEM((TI, 2 * D), jnp.float32),
            pltpu.VMEM((TI, 2 * D), jnp.float32),
            pltpu.VMEM((CH_G, 2 * D), jnp.float32),
            pltpu.SemaphoreType.DMA,
            pltpu.SemaphoreType.DMA,
        ],
    )
    return fn(pq, row2d, col2d)


def _edge_body(s_ref, eat_ref, w0_ref, w1_ref, w2_ref,
               s0_ref, t0_ref, b1_ref, s1_ref, t1_ref, b2_ref, s2_ref, t2_ref,
               et_ref, ew_ref):
    st = jnp.transpose(s_ref[:, 0:D])
    h = w0_ref[...] @ eat_ref[...] + st
    h = jnp.maximum(h, 0.0) * s0_ref[...] + t0_ref[...]
    h = w1_ref[...] @ h + b1_ref[...]
    h = jnp.maximum(h, 0.0) * s1_ref[...] + t1_ref[...]
    h = w2_ref[...] @ h + b2_ref[...]
    h = jnp.maximum(h, 0.0) * s2_ref[...] + t2_ref[...]
    et_ref[...] = h
    hr = jnp.transpose(h)
    ew_ref[:, 0:D] = hr
    ew_ref[:, D:D + 16] = jnp.full((hr.shape[0], 16), 1.0, jnp.float32)
    ew_ref[:, D + 16:2 * D] = jnp.zeros((hr.shape[0], 48), jnp.float32)


def _edge_mlp(s, ea_t, chunk, w0t, w1t, w2t, s0, t0, b1, s1, t1, b2, s2, t2):
    blk = 3200
    grid = EH // blk
    off = chunk * grid
    s_spec = pl.BlockSpec((blk, 2 * D), lambda i: (i, 0))
    ea_spec = pl.BlockSpec((D, blk), lambda i: (0, i + off))
    t_spec = pl.BlockSpec((D, blk), lambda i: (0, i))
    w_spec = pl.BlockSpec((D, D), lambda i: (0, 0))
    v_spec = pl.BlockSpec((D, 1), lambda i: (0, 0))
    ew_spec = pl.BlockSpec((blk, 2 * D), lambda i: (i, 0))
    return pl.pallas_call(
        _edge_body,
        grid=(grid,),
        in_specs=[s_spec, ea_spec,
                  w_spec, w_spec, w_spec,
                  v_spec, v_spec, v_spec, v_spec, v_spec, v_spec, v_spec,
                  v_spec],
        out_specs=[t_spec, ew_spec],
        out_shape=[jax.ShapeDtypeStruct((D, EH), jnp.float32),
                   jax.ShapeDtypeStruct((EH, 2 * D), jnp.float32)],
    )(s, ea_t, w0t, w1t, w2t, s0, t0, b1, s1, t1, b2, s2, t2)


CH_S = 125


def _scatter_body(ew_hbm, row_hbm, zn_hbm, sums_hbm,
                  ridx_v, rows_v, rows1_v, shared_sum, sem0, sem1):
    cid = lax.axis_index("c")
    sid = lax.axis_index("s")
    wid = sid * NC + cid
    npt = N // NS

    pltpu.sync_copy(zn_hbm.at[pl.ds(sid * npt, npt)],
                    shared_sum.at[pl.ds(sid * npt, npt)])
    plsc.subcore_barrier()

    nq = CHUNK // CH_S

    def load(t, q, slot):
        eoff = wid * EPW + t * CHUNK + q * CH_S
        buf, sem = (rows_v, sem0) if slot == 0 else (rows1_v, sem1)
        return pltpu.async_copy(ew_hbm.at[pl.ds(eoff, CH_S)], buf, sem)

    def outer(t, carry):
        irow = wid * IPW + t * IN_PER_OUTER
        pltpu.sync_copy(row_hbm.at[pl.ds(irow, IN_PER_OUTER)], ridx_v)
        inflight = load(t, 0, 0)
        for q in range(nq):
            slot = q % 2
            cur = inflight
            if q + 1 < nq:
                inflight = load(t, q + 1, (q + 1) % 2)
            cur.wait()
            buf = rows_v if slot == 0 else rows1_v
            for j in range(CH_S // TI):
                jj = q * (CH_S // TI) + j
                pltpu.sync_copy(buf.at[pl.ds(j * TI, TI)],
                                shared_sum.at[ridx_v.at[jj]], add=True)
        return carry

    lax.fori_loop(0, OUTER, outer, 0)
    plsc.subcore_barrier()

    pltpu.sync_copy(shared_sum.at[pl.ds(sid * npt, npt)],
                    sums_hbm.at[pl.ds(cid * N + sid * npt, npt)])


def _sc_scatter(ewide, row2d, zn):
    fn = pl.kernel(
        _scatter_body,
        out_type=jax.ShapeDtypeStruct((NC * N, 2 * D), jnp.float32),
        compiler_params=pltpu.CompilerParams(use_tc_tiling_on_sc=False),
        mesh=plsc.VectorSubcoreMesh(core_axis_name="c", subcore_axis_name="s"),
        scratch_types=[
            pltpu.VMEM((IN_PER_OUTER, TI), jnp.int32),
            pltpu.VMEM((CH_S, 2 * D), jnp.float32),
            pltpu.VMEM((CH_S, 2 * D), jnp.float32),
            pltpu.VMEM_SHARED((N, 2 * D), jnp.float32),
            pltpu.SemaphoreType.DMA,
            pltpu.SemaphoreType.DMA,
        ],
    )
    return fn(ewide, row2d, zn)


def _final_body(x_ref, u_ref, batch2_ref, batchr_ref,
                sa_ref, sb_ref, sc_ref, sd_ref,
                nwx_ref, nwv_ref, nwu_ref, nw1_ref, nw2_ref,
                nb0_ref, ns0_ref, nt0_ref, nb1_ref, ns1_ref, nt1_ref,
                nb2_ref, ns2_ref, nt2_ref,
                gwe_ref, gwv_ref, gwu_ref, gw1_ref, gw2_ref,
                gb0_ref, gs0_ref, gt0_ref, gb1_ref, gs1_ref, gt1_ref,
                gb2_ref, gs2_ref, gt2_ref,
                xout_ref, uout_ref):
    acc = (sa_ref[...] + sb_ref[...]) + (sc_ref[...] + sd_ref[...])
    cnt = acc[:, D:D + 1]
    v_e = acc[:, 0:D] / jnp.maximum(cnt, 1.0)
    x = x_ref[...]
    uu = u_ref[...]
    m = (batch2_ref[...] == lax.broadcasted_iota(jnp.int32, (1, B), 1))
    m = m.astype(jnp.float32)
    mt = (lax.broadcasted_iota(jnp.int32, (B, 1), 0) == batchr_ref[0:1, :])
    mt = mt.astype(jnp.float32)

    h = x @ nwx_ref[...] + v_e @ nwv_ref[...] + m @ (uu @ nwu_ref[...])
    h = h + nb0_ref[...]
    h = jnp.maximum(h, 0.0) * ns0_ref[...] + nt0_ref[...]
    h = h @ nw1_ref[...] + nb1_ref[...]
    h = jnp.maximum(h, 0.0) * ns1_ref[...] + nt1_ref[...]
    h = h @ nw2_ref[...] + nb2_ref[...]
    h = jnp.maximum(h, 0.0) * ns2_ref[...] + nt2_ref[...]
    xout_ref[...] = h

    cnt_g = jnp.maximum(jnp.sum(mt, axis=1, keepdims=True), 1.0)
    u_e = (mt @ v_e) / cnt_g
    u_v = (mt @ h) / cnt_g
    g = u_e @ gwe_ref[...] + u_v @ gwv_ref[...] + uu @ gwu_ref[...]
    g = g + gb0_ref[...]
    g = jnp.maximum(g, 0.0) * gs0_ref[...] + gt0_ref[...]
    g = g @ gw1_ref[...] + gb1_ref[...]
    g = jnp.maximum(g, 0.0) * gs1_ref[...] + gt1_ref[...]
    g = g @ gw2_ref[...] + gb2_ref[...]
    g = jnp.maximum(g, 0.0) * gs2_ref[...] + gt2_ref[...]
    uout_ref[...] = g


def _final(args):
    return pl.pallas_call(
        _final_body,
        out_shape=[jax.ShapeDtypeStruct((N, D), jnp.float32),
                   jax.ShapeDtypeStruct((B, D), jnp.float32)],
    )(*args)


def kernel(x, edge_attr, u, edge_index, batch,
           eW0, eWr, eb, eg, ebe,
           nW0, nWr, nb, ng, nbe,
           gW0, gWr, gb, gg, gbe):
    row = edge_index[0]
    col = edge_index[1]
    batch2 = batch[:, None]
    batchr = jnp.broadcast_to(batch[None, :], (8, N))

    pq = _prep(x, u, batch2,
               eW0[0:D], eW0[D:2 * D], eW0[3 * D:4 * D],
               eb[0].reshape(1, D))

    def r1(v):
        return v.reshape(1, D)

    def c1(v):
        return v.reshape(D, 1)
    row2d = row.reshape(E // TI, TI)
    col2d = col.reshape(E // TI, TI)
    ea_t = edge_attr.T
    zn = jnp.zeros((N, 2 * D), jnp.float32)
    irpc = EH // TI
    ets, sums_list = [], []
    for c in range(NCH):
        r2 = row2d[c * irpc:(c + 1) * irpc]
        c2 = col2d[c * irpc:(c + 1) * irpc]
        s = _sc_gather(pq, r2, c2)
        e_t, ewide = _edge_mlp(
            s, ea_t, c,
            eW0[2 * D:3 * D].T, eWr[0].T, eWr[1].T,
            c1(eg[0] * _EPS_SCALE), c1(ebe[0]),
            c1(eb[1]), c1(eg[1] * _EPS_SCALE), c1(ebe[1]),
            c1(eb[2]), c1(eg[2] * _EPS_SCALE), c1(ebe[2]))
        ets.append(e_t)
        sums_list.append(_sc_scatter(ewide, r2, zn))
    e_out = jnp.concatenate(ets, axis=1).T

    x_out, u_out = _final([
        x, u, batch2, batchr,
        sums_list[0][0:N], sums_list[0][N:2 * N],
        sums_list[1][0:N], sums_list[1][N:2 * N],
        nW0[0:D], nW0[D:2 * D], nW0[2 * D:3 * D], nWr[0], nWr[1],
        r1(nb[0]), r1(ng[0] * _EPS_SCALE), r1(nbe[0]),
        r1(nb[1]), r1(ng[1] * _EPS_SCALE), r1(nbe[1]),
        r1(nb[2]), r1(ng[2] * _EPS_SCALE), r1(nbe[2]),
        gW0[0:D], gW0[D:2 * D], gW0[2 * D:3 * D], gWr[0], gWr[1],
        r1(gb[0]), r1(gg[0] * _EPS_SCALE), r1(gbe[0]),
        r1(gb[1]), r1(gg[1] * _EPS_SCALE), r1(gbe[1]),
        r1(gb[2]), r1(gg[2] * _EPS_SCALE), r1(gbe[2]),
    ])
    return x_out, e_out, u_out
